# baseline (speedup 1.0000x reference)
import functools

import jax
import jax.numpy as jnp
from jax import lax
from jax.experimental import pallas as pl
from jax.experimental.pallas import tpu as pltpu
from jax.experimental.pallas import tpu_sc as plsc

_NC = 2
_NS = 16
_NW = _NC * _NS
_EU = 128


def _units_per_tile(e, nbuf):
    units = -(-e // _EU)
    nu = -(-units // _NW)
    return -(-nu // nbuf) * nbuf


def _make_seg_sum(n, e, dp, nbuf, name):
    nu = _units_per_tile(e, nbuf)
    npad = -(-n // _EU) * _EU
    rpt = npad // _NS
    mesh = plsc.VectorSubcoreMesh(core_axis_name="c", subcore_axis_name="s")

    @functools.partial(
        pl.kernel,
        out_type=jax.ShapeDtypeStruct((_NC, npad, dp), jnp.float32),
        mesh=mesh,
        scratch_types=(
            [pltpu.VMEM((nu, _EU), jnp.int32),
             pltpu.VMEM((nu, _EU), jnp.int32)]
            + [pltpu.VMEM((_EU, dp), jnp.float32) for _ in range(nbuf)]
            + [pltpu.SemaphoreType.DMA for _ in range(2 * nbuf)]
            + [pltpu.VMEM_SHARED((npad, dp), jnp.float32)]
        ),
        compiler_params=pltpu.CompilerParams(use_tc_tiling_on_sc=False),
        name=name,
    )
    def seg_sum(feat_hbm, src_hbm, dst_hbm, zero_hbm, out_hbm, sidx, didx,
                *rest):
        rows = rest[:nbuf]
        gsem = rest[nbuf:2 * nbuf]
        ssem = rest[2 * nbuf:3 * nbuf]
        acc = rest[3 * nbuf]
        c = lax.axis_index("c")
        s = lax.axis_index("s")
        w = s * _NC + c
        pltpu.sync_copy(zero_hbm, acc.at[pl.ds(s * rpt, rpt)])
        pltpu.sync_copy(src_hbm.at[pl.ds(w * nu, nu)], sidx)
        pltpu.sync_copy(dst_hbm.at[pl.ds(w * nu, nu)], didx)
        plsc.subcore_barrier()

        def gather(j, r):
            pltpu.async_copy(feat_hbm.at[sidx.at[j]], rows[r], gsem[r])

        def scatter(j, r):
            pltpu.async_copy(rows[r], acc.at[didx.at[j]], ssem[r], add=True)

        def wait_g(r):
            pltpu.make_async_copy(feat_hbm.at[sidx.at[0]], rows[r],
                                  gsem[r]).wait()

        def wait_s(r):
            pltpu.make_async_copy(rows[r], acc.at[didx.at[0]], ssem[r]).wait()

        for r in range(nbuf):
            gather(r, r)
        nblk = nu // nbuf

        def body(b, carry):
            j0 = b * nbuf
            for r in range(nbuf):
                wait_g(r)
                scatter(j0 + r, r)
            for r in range(nbuf):
                wait_s(r)
                gather(j0 + nbuf + r, r)
            return carry

        lax.fori_loop(0, nblk - 1, body, 0)
        j0 = (nblk - 1) * nbuf
        for r in range(nbuf):
            wait_g(r)
            scatter(j0 + r, r)
        for r in range(nbuf):
            wait_s(r)
        plsc.subcore_barrier()
        pltpu.sync_copy(acc.at[pl.ds(s * rpt, rpt)],
                        out_hbm.at[c, pl.ds(s * rpt, rpt)])

    return seg_sum


def _tc_dense(x, sum1a, sum1b, w1l, w1r, w2lp, w2rp, b1, b2p, n, blk):
    d = x.shape[1]
    grid = (n // blk,)

    dh = d // 2

    def body(x_ref, sa_ref, sb_ref, w1l_ref, w1r_ref, w2l_ref, w2r_ref,
             b1_ref, b2_ref, y2_ref, hr_ref, rc_ref):
        sa = sa_ref[0] + sa_ref[1]
        sb = sb_ref[0] + sb_ref[1]
        agg = jnp.concatenate([sa[:, :dh], sb], axis=1)
        cnt = sa[:, dh:dh + 1]
        recip = 1.0 / jnp.maximum(cnt, 1.0)
        aggm = agg * recip
        h = aggm @ w1l_ref[...].T + x_ref[...] @ w1r_ref[...].T + b1_ref[...]
        h = jnp.maximum(h, 0.0)
        y2_ref[...] = h @ w2l_ref[...].T
        hr_ref[...] = h @ w2r_ref[...].T + b2_ref[...]
        rc_ref[...] = jnp.broadcast_to(recip, (blk, 16))

    dpa, dpb = sum1a.shape[2], sum1b.shape[2]
    return pl.pallas_call(
        body,
        grid=grid,
        in_specs=[
            pl.BlockSpec((blk, d), lambda i: (i, 0)),
            pl.BlockSpec((2, blk, dpa), lambda i: (0, i, 0)),
            pl.BlockSpec((2, blk, dpb), lambda i: (0, i, 0)),
            pl.BlockSpec(w1l.shape, lambda i: (0, 0)),
            pl.BlockSpec(w1r.shape, lambda i: (0, 0)),
            pl.BlockSpec(w2lp.shape, lambda i: (0, 0)),
            pl.BlockSpec(w2rp.shape, lambda i: (0, 0)),
            pl.BlockSpec(b1.shape, lambda i: (0, 0)),
            pl.BlockSpec(b2p.shape, lambda i: (0, 0)),
        ],
        out_specs=[
            pl.BlockSpec((blk, 16), lambda i: (i, 0)),
            pl.BlockSpec((blk, 16), lambda i: (i, 0)),
            pl.BlockSpec((blk, 16), lambda i: (i, 0)),
        ],
        out_shape=[
            jax.ShapeDtypeStruct((n, 16), jnp.float32),
            jax.ShapeDtypeStruct((n, 16), jnp.float32),
            jax.ShapeDtypeStruct((n, 16), jnp.float32),
        ],
    )(x, sum1a, sum1b, w1l, w1r, w2lp, w2rp, b1, b2p)


def _tc_combine(sum2, rc16, hr, n):
    def body(s2_ref, rc_ref, hr_ref, o_ref):
        o_ref[...] = (s2_ref[0] + s2_ref[1]) * rc_ref[...] + hr_ref[...]

    return pl.pallas_call(
        body,
        grid=(1,),
        in_specs=[
            pl.BlockSpec((2, n, 16), lambda i: (0, 0, 0)),
            pl.BlockSpec((n, 16), lambda i: (0, 0)),
            pl.BlockSpec((n, 16), lambda i: (0, 0)),
        ],
        out_specs=pl.BlockSpec((n, 16), lambda i: (0, 0)),
        out_shape=jax.ShapeDtypeStruct((n, 16), jnp.float32),
    )(sum2, rc16, hr)


def kernel(x, edge_index, W1l, b1, W1r, W2l, b2, W2r):
    n, d = x.shape
    e = edge_index.shape[1]
    o = W2l.shape[0]

    nbufa, nbufb, nbuf2 = 5, 8, 8
    dh = d // 2
    dpa = dh + 16

    nu = _units_per_tile(e, nbufa)
    assert nu == _units_per_tile(e, nbufb) == _units_per_tile(e, nbuf2)
    tot = nu * _NW * _EU
    npad = -(-n // _EU) * _EU
    src = jnp.concatenate(
        [edge_index[0], jnp.zeros((tot - e,), jnp.int32)]).reshape(-1, _EU)
    dst = jnp.concatenate(
        [edge_index[1],
         n + jnp.arange(tot - e, dtype=jnp.int32) % (npad - n)]).reshape(-1, _EU)

    xxa = jnp.concatenate([x[:, :dh], jnp.ones((n, 1), jnp.float32),
                           jnp.zeros((n, 15), jnp.float32)], axis=1)
    xxb = x[:, dh:]
    npad = -(-n // _EU) * _EU
    zeroa = jnp.zeros((npad // _NS, dpa), jnp.float32)
    zerob = jnp.zeros((npad // _NS, dh), jnp.float32)
    zero2 = jnp.zeros((npad // _NS, 16), jnp.float32)
    w2lp = jnp.zeros((16, d), jnp.float32).at[:o].set(W2l)
    w2rp = jnp.zeros((16, d), jnp.float32).at[:o].set(W2r)
    b2p = jnp.zeros((1, 16), jnp.float32).at[0, :o].set(b2)
    b1r = b1.reshape(1, d)

    seg1a = _make_seg_sum(n, e, dpa, nbufa, "sage_seg_sum_l1a")
    seg1b = _make_seg_sum(n, e, dh, nbufb, "sage_seg_sum_l1b")
    seg2 = _make_seg_sum(n, e, 16, nbuf2, "sage_seg_sum_l2")

    sum1a = seg1a(xxa, src, dst, zeroa)
    sum1b = seg1b(xxb, src, dst, zerob)
    y2, hr, rc16 = _tc_dense(x, sum1a, sum1b, W1l, W1r, w2lp, w2rp, b1r, b2p,
                             n, 1000)
    sum2 = seg2(y2, src, dst, zero2)
    outp = _tc_combine(sum2, rc16, hr, n)
    return outp[:, :o]


# speedup vs baseline: 2.7484x; 2.7484x over previous
import functools

import jax
import jax.numpy as jnp
from jax import lax
from jax.experimental import pallas as pl
from jax.experimental.pallas import tpu as pltpu
from jax.experimental.pallas import tpu_sc as plsc

_NC = 2
_NS = 16
_NW = _NC * _NS
_EU = 128


def _units_per_tile(e, nbuf):
    units = -(-e // _EU)
    nu = -(-units // _NW)
    return -(-nu // nbuf) * nbuf


def _make_seg_sum(n, e, dp, nbuf, name):
    nu = _units_per_tile(e, nbuf)
    npad = -(-n // _EU) * _EU
    rpt = npad // _NS
    mesh = plsc.VectorSubcoreMesh(core_axis_name="c", subcore_axis_name="s")

    @functools.partial(
        pl.kernel,
        out_type=jax.ShapeDtypeStruct((_NC, npad, dp), jnp.float32),
        mesh=mesh,
        scratch_types=(
            [pltpu.VMEM((nu, _EU), jnp.int32),
             pltpu.VMEM((nu, _EU), jnp.int32)]
            + [pltpu.VMEM((_EU, dp), jnp.float32) for _ in range(nbuf)]
            + [pltpu.SemaphoreType.DMA for _ in range(2 * nbuf)]
            + [pltpu.VMEM_SHARED((npad, dp), jnp.float32)]
        ),
        compiler_params=pltpu.CompilerParams(use_tc_tiling_on_sc=False),
        name=name,
    )
    def seg_sum(feat_hbm, src_hbm, dst_hbm, zero_hbm, out_hbm, sidx, didx,
                *rest):
        rows = rest[:nbuf]
        gsem = rest[nbuf:2 * nbuf]
        ssem = rest[2 * nbuf:3 * nbuf]
        acc = rest[3 * nbuf]
        c = lax.axis_index("c")
        s = lax.axis_index("s")
        w = s * _NC + c
        pltpu.sync_copy(zero_hbm, acc.at[pl.ds(s * rpt, rpt)])
        pltpu.sync_copy(src_hbm.at[pl.ds(w * nu, nu)], sidx)
        pltpu.sync_copy(dst_hbm.at[pl.ds(w * nu, nu)], didx)
        plsc.subcore_barrier()

        def gather(j, r):
            pltpu.async_copy(feat_hbm.at[sidx.at[j]], rows[r], gsem[r])

        def scatter(j, r):
            pltpu.async_copy(rows[r], acc.at[didx.at[j]], ssem[r], add=True)

        def wait_g(r):
            pltpu.make_async_copy(feat_hbm.at[sidx.at[0]], rows[r],
                                  gsem[r]).wait()

        def wait_s(r):
            pltpu.make_async_copy(rows[r], acc.at[didx.at[0]], ssem[r]).wait()

        for r in range(nbuf):
            gather(r, r)
        nblk = nu // nbuf

        def body(b, carry):
            j0 = b * nbuf
            for r in range(nbuf):
                wait_g(r)
                scatter(j0 + r, r)
            for r in range(nbuf):
                wait_s(r)
                gather(j0 + nbuf + r, r)
            return carry

        lax.fori_loop(0, nblk - 1, body, 0)
        j0 = (nblk - 1) * nbuf
        for r in range(nbuf):
            wait_g(r)
            scatter(j0 + r, r)
        for r in range(nbuf):
            wait_s(r)
        plsc.subcore_barrier()
        pltpu.sync_copy(acc.at[pl.ds(s * rpt, rpt)],
                        out_hbm.at[c, pl.ds(s * rpt, rpt)])

    return seg_sum


def _tc_dense(x, sum1a, sum1b, w1l, w1r, w2lp, w2rp, b1, b2p, n, blk):
    d = x.shape[1]
    grid = (n // blk,)

    dh = d // 2

    def body(x_ref, sa_ref, sb_ref, w1l_ref, w1r_ref, w2l_ref, w2r_ref,
             b1_ref, b2_ref, y2_ref, hr_ref, rc_ref):
        sa = sa_ref[0] + sa_ref[1]
        sb = sb_ref[0] + sb_ref[1]
        agg = jnp.concatenate([sa[:, :dh], sb], axis=1)
        cnt = sa[:, dh:dh + 1]
        recip = 1.0 / jnp.maximum(cnt, 1.0)
        aggm = agg * recip
        h = aggm @ w1l_ref[...].T + x_ref[...] @ w1r_ref[...].T + b1_ref[...]
        h = jnp.maximum(h, 0.0)
        y2_ref[...] = h @ w2l_ref[...].T
        hr_ref[...] = h @ w2r_ref[...].T + b2_ref[...]
        rc_ref[...] = jnp.broadcast_to(recip, (blk, 16))

    dpa, dpb = sum1a.shape[2], sum1b.shape[2]
    return pl.pallas_call(
        body,
        grid=grid,
        in_specs=[
            pl.BlockSpec((blk, d), lambda i: (i, 0)),
            pl.BlockSpec((2, blk, dpa), lambda i: (0, i, 0)),
            pl.BlockSpec((2, blk, dpb), lambda i: (0, i, 0)),
            pl.BlockSpec(w1l.shape, lambda i: (0, 0)),
            pl.BlockSpec(w1r.shape, lambda i: (0, 0)),
            pl.BlockSpec(w2lp.shape, lambda i: (0, 0)),
            pl.BlockSpec(w2rp.shape, lambda i: (0, 0)),
            pl.BlockSpec(b1.shape, lambda i: (0, 0)),
            pl.BlockSpec(b2p.shape, lambda i: (0, 0)),
        ],
        out_specs=[
            pl.BlockSpec((blk, 16), lambda i: (i, 0)),
            pl.BlockSpec((blk, 16), lambda i: (i, 0)),
            pl.BlockSpec((blk, 16), lambda i: (i, 0)),
        ],
        out_shape=[
            jax.ShapeDtypeStruct((n, 16), jnp.float32),
            jax.ShapeDtypeStruct((n, 16), jnp.float32),
            jax.ShapeDtypeStruct((n, 16), jnp.float32),
        ],
    )(x, sum1a, sum1b, w1l, w1r, w2lp, w2rp, b1, b2p)


def _tc_combine(sum2, rc16, hr, n):
    def body(s2_ref, rc_ref, hr_ref, o_ref):
        o_ref[...] = (s2_ref[0] + s2_ref[1]) * rc_ref[...] + hr_ref[...]

    return pl.pallas_call(
        body,
        grid=(1,),
        in_specs=[
            pl.BlockSpec((2, n, 16), lambda i: (0, 0, 0)),
            pl.BlockSpec((n, 16), lambda i: (0, 0)),
            pl.BlockSpec((n, 16), lambda i: (0, 0)),
        ],
        out_specs=pl.BlockSpec((n, 16), lambda i: (0, 0)),
        out_shape=jax.ShapeDtypeStruct((n, 16), jnp.float32),
    )(sum2, rc16, hr)


def kernel(x, edge_index, W1l, b1, W1r, W2l, b2, W2r):
    n, d = x.shape
    e = edge_index.shape[1]
    o = W2l.shape[0]

    nbufa, nbufb, nbuf2 = 5, 8, 8
    dh = d // 2
    dpa = dh + 16

    nu = _units_per_tile(e, nbufa)
    assert nu == _units_per_tile(e, nbufb) == _units_per_tile(e, nbuf2)
    tot = nu * _NW * _EU
    npad = -(-n // _EU) * _EU
    src = jnp.concatenate(
        [edge_index[0],
         jnp.arange(tot - e, dtype=jnp.int32) % n]).reshape(-1, _EU)
    dst = jnp.concatenate(
        [edge_index[1],
         n + jnp.arange(tot - e, dtype=jnp.int32) % (npad - n)]).reshape(-1, _EU)

    xxa = jnp.concatenate([x[:, :dh], jnp.ones((n, 1), jnp.float32),
                           jnp.zeros((n, 15), jnp.float32)], axis=1)
    xxb = x[:, dh:]
    npad = -(-n // _EU) * _EU
    zeroa = jnp.zeros((npad // _NS, dpa), jnp.float32)
    zerob = jnp.zeros((npad // _NS, dh), jnp.float32)
    zero2 = jnp.zeros((npad // _NS, 16), jnp.float32)
    w2lp = jnp.zeros((16, d), jnp.float32).at[:o].set(W2l)
    w2rp = jnp.zeros((16, d), jnp.float32).at[:o].set(W2r)
    b2p = jnp.zeros((1, 16), jnp.float32).at[0, :o].set(b2)
    b1r = b1.reshape(1, d)

    seg1a = _make_seg_sum(n, e, dpa, nbufa, "sage_seg_sum_l1a")
    seg1b = _make_seg_sum(n, e, dh, nbufb, "sage_seg_sum_l1b")
    seg2 = _make_seg_sum(n, e, 16, nbuf2, "sage_seg_sum_l2")

    sum1a = seg1a(xxa, src, dst, zeroa)
    sum1b = seg1b(xxb, src, dst, zerob)
    y2, hr, rc16 = _tc_dense(x, sum1a, sum1b, W1l, W1r, w2lp, w2rp, b1r, b2p,
                             n, 1000)
    sum2 = seg2(y2, src, dst, zero2)
    outp = _tc_combine(sum2, rc16, hr, n)
    return outp[:, :o]


# speedup vs baseline: 2.7514x; 1.0011x over previous
import functools

import jax
import jax.numpy as jnp
from jax import lax
from jax.experimental import pallas as pl
from jax.experimental.pallas import tpu as pltpu
from jax.experimental.pallas import tpu_sc as plsc

_NC = 2
_NS = 16
_NW = _NC * _NS
_EU = 128


def _units_per_tile(e, nbuf):
    units = -(-e // _EU)
    nu = -(-units // _NW)
    return -(-nu // nbuf) * nbuf


def _make_seg_sum(n, e, dp, nbuf, name):
    nu = _units_per_tile(e, nbuf)
    npad = -(-n // _EU) * _EU
    rpt = npad // _NS
    mesh = plsc.VectorSubcoreMesh(core_axis_name="c", subcore_axis_name="s")

    @functools.partial(
        pl.kernel,
        out_type=jax.ShapeDtypeStruct((_NC, npad, dp), jnp.float32),
        mesh=mesh,
        scratch_types=(
            [pltpu.VMEM((nu, _EU), jnp.int32),
             pltpu.VMEM((nu, _EU), jnp.int32)]
            + [pltpu.VMEM((_EU, dp), jnp.float32) for _ in range(nbuf)]
            + [pltpu.SemaphoreType.DMA for _ in range(2 * nbuf)]
            + [pltpu.VMEM_SHARED((npad, dp), jnp.float32)]
        ),
        compiler_params=pltpu.CompilerParams(use_tc_tiling_on_sc=False),
        name=name,
    )
    def seg_sum(feat_hbm, src_hbm, dst_hbm, zero_hbm, out_hbm, sidx, didx,
                *rest):
        rows = rest[:nbuf]
        gsem = rest[nbuf:2 * nbuf]
        ssem = rest[2 * nbuf:3 * nbuf]
        acc = rest[3 * nbuf]
        c = lax.axis_index("c")
        s = lax.axis_index("s")
        w = s * _NC + c
        pltpu.sync_copy(zero_hbm, acc.at[pl.ds(s * rpt, rpt)])
        pltpu.sync_copy(src_hbm.at[pl.ds(w * nu, nu)], sidx)
        pltpu.sync_copy(dst_hbm.at[pl.ds(w * nu, nu)], didx)
        plsc.subcore_barrier()

        def gather(j, r):
            pltpu.async_copy(feat_hbm.at[sidx.at[j]], rows[r], gsem[r])

        def scatter(j, r):
            pltpu.async_copy(rows[r], acc.at[didx.at[j]], ssem[r], add=True)

        def wait_g(r):
            pltpu.make_async_copy(feat_hbm.at[sidx.at[0]], rows[r],
                                  gsem[r]).wait()

        def wait_s(r):
            pltpu.make_async_copy(rows[r], acc.at[didx.at[0]], ssem[r]).wait()

        for r in range(nbuf):
            gather(r, r)
        nblk = nu // nbuf

        def body(b, carry):
            j0 = b * nbuf
            for r in range(nbuf):
                wait_g(r)
                scatter(j0 + r, r)
            for r in range(nbuf):
                wait_s(r)
                gather(j0 + nbuf + r, r)
            return carry

        lax.fori_loop(0, nblk - 1, body, 0)
        j0 = (nblk - 1) * nbuf
        for r in range(nbuf):
            wait_g(r)
            scatter(j0 + r, r)
        for r in range(nbuf):
            wait_s(r)
        plsc.subcore_barrier()
        pltpu.sync_copy(acc.at[pl.ds(s * rpt, rpt)],
                        out_hbm.at[c, pl.ds(s * rpt, rpt)])

    return seg_sum


def _tc_dense(x, sum1a, sum1b, w1l, w1r, w2lp, w2rp, b1, b2p, n, blk):
    d = x.shape[1]
    grid = (n // blk,)

    dh = d // 2

    def body(x_ref, sa_ref, sb_ref, w1l_ref, w1r_ref, w2l_ref, w2r_ref,
             b1_ref, b2_ref, y2_ref, hr_ref, rc_ref):
        sa = sa_ref[0] + sa_ref[1]
        sb = sb_ref[0] + sb_ref[1]
        agg = jnp.concatenate([sa[:, :dh], sb], axis=1)
        cnt = sa[:, dh:dh + 1]
        recip = 1.0 / jnp.maximum(cnt, 1.0)
        aggm = agg * recip
        h = aggm @ w1l_ref[...].T + x_ref[...] @ w1r_ref[...].T + b1_ref[...]
        h = jnp.maximum(h, 0.0)
        y2_ref[...] = h @ w2l_ref[...].T
        hr_ref[...] = h @ w2r_ref[...].T + b2_ref[...]
        rc_ref[...] = jnp.broadcast_to(recip, (blk, 16))

    dpa, dpb = sum1a.shape[2], sum1b.shape[2]
    return pl.pallas_call(
        body,
        grid=grid,
        in_specs=[
            pl.BlockSpec((blk, d), lambda i: (i, 0)),
            pl.BlockSpec((2, blk, dpa), lambda i: (0, i, 0)),
            pl.BlockSpec((2, blk, dpb), lambda i: (0, i, 0)),
            pl.BlockSpec(w1l.shape, lambda i: (0, 0)),
            pl.BlockSpec(w1r.shape, lambda i: (0, 0)),
            pl.BlockSpec(w2lp.shape, lambda i: (0, 0)),
            pl.BlockSpec(w2rp.shape, lambda i: (0, 0)),
            pl.BlockSpec(b1.shape, lambda i: (0, 0)),
            pl.BlockSpec(b2p.shape, lambda i: (0, 0)),
        ],
        out_specs=[
            pl.BlockSpec((blk, 16), lambda i: (i, 0)),
            pl.BlockSpec((blk, 16), lambda i: (i, 0)),
            pl.BlockSpec((blk, 16), lambda i: (i, 0)),
        ],
        out_shape=[
            jax.ShapeDtypeStruct((n, 16), jnp.float32),
            jax.ShapeDtypeStruct((n, 16), jnp.float32),
            jax.ShapeDtypeStruct((n, 16), jnp.float32),
        ],
    )(x, sum1a, sum1b, w1l, w1r, w2lp, w2rp, b1, b2p)


def _tc_combine(sum2, rc16, hr, n, o):
    def body(s2_ref, rc_ref, hr_ref, o_ref):
        o_ref[...] = ((s2_ref[0] + s2_ref[1]) * rc_ref[...] + hr_ref[...])[:, :o]

    return pl.pallas_call(
        body,
        grid=(1,),
        in_specs=[
            pl.BlockSpec((2, n, 16), lambda i: (0, 0, 0)),
            pl.BlockSpec((n, 16), lambda i: (0, 0)),
            pl.BlockSpec((n, 16), lambda i: (0, 0)),
        ],
        out_specs=pl.BlockSpec((n, o), lambda i: (0, 0)),
        out_shape=jax.ShapeDtypeStruct((n, o), jnp.float32),
    )(sum2, rc16, hr)


def kernel(x, edge_index, W1l, b1, W1r, W2l, b2, W2r):
    n, d = x.shape
    e = edge_index.shape[1]
    o = W2l.shape[0]

    nbufa, nbufb, nbuf2 = 5, 8, 10
    dh = d // 2
    dpa = dh + 16

    nu = _units_per_tile(e, nbufa)
    assert nu == _units_per_tile(e, nbufb) == _units_per_tile(e, nbuf2)
    tot = nu * _NW * _EU
    npad = -(-n // _EU) * _EU
    src = jnp.concatenate(
        [edge_index[0],
         jnp.arange(tot - e, dtype=jnp.int32) % n]).reshape(-1, _EU)
    dst = jnp.concatenate(
        [edge_index[1],
         n + jnp.arange(tot - e, dtype=jnp.int32) % (npad - n)]).reshape(-1, _EU)

    xxa = jnp.concatenate([x[:, :dh], jnp.ones((n, 1), jnp.float32),
                           jnp.zeros((n, 15), jnp.float32)], axis=1)
    xxb = x[:, dh:]
    npad = -(-n // _EU) * _EU
    zeroa = jnp.zeros((npad // _NS, dpa), jnp.float32)
    zerob = jnp.zeros((npad // _NS, dh), jnp.float32)
    zero2 = jnp.zeros((npad // _NS, 16), jnp.float32)
    w2lp = jnp.zeros((16, d), jnp.float32).at[:o].set(W2l)
    w2rp = jnp.zeros((16, d), jnp.float32).at[:o].set(W2r)
    b2p = jnp.zeros((1, 16), jnp.float32).at[0, :o].set(b2)
    b1r = b1.reshape(1, d)

    seg1a = _make_seg_sum(n, e, dpa, nbufa, "sage_seg_sum_l1a")
    seg1b = _make_seg_sum(n, e, dh, nbufb, "sage_seg_sum_l1b")
    seg2 = _make_seg_sum(n, e, 16, nbuf2, "sage_seg_sum_l2")

    sum1a = seg1a(xxa, src, dst, zeroa)
    sum1b = seg1b(xxb, src, dst, zerob)
    y2, hr, rc16 = _tc_dense(x, sum1a, sum1b, W1l, W1r, w2lp, w2rp, b1r, b2p,
                             n, 1000)
    sum2 = seg2(y2, src, dst, zero2)
    return _tc_combine(sum2, rc16, hr, n, o)
